# Initial kernel scaffold; baseline (speedup 1.0000x reference)
import jax
import jax.numpy as jnp
from jax.experimental import pallas as pl


def kernel(x, sim_matrix, gates):
    raise NotImplementedError("write your pallas kernel here")


# speedup vs baseline: 1.3611x; 1.3611x over previous
import functools

import jax
import jax.numpy as jnp
from jax.experimental import pallas as pl

EXPERTS = 8
BLOCK_T = 2048


def _gate_body(x_ref, w_ref, g_ref, o_ref):
    xb = x_ref[...]
    w = w_ref[...]
    wnorm = jnp.sqrt(jnp.sum(w * w, axis=0, keepdims=True))
    wn = w / jnp.maximum(wnorm, 1e-12)
    dot = jnp.dot(xb, wn, preferred_element_type=jnp.float32)
    rnorm = jnp.sqrt(jnp.sum(xb * xb, axis=1, keepdims=True))
    z = dot / jnp.maximum(rnorm, 1e-12)
    g = jax.nn.sigmoid(g_ref[...])
    o_ref[...] = jnp.maximum(jax.nn.sigmoid(z) - g, 0.0)


@functools.partial(jax.jit, static_argnames=())
def kernel(x, sim_matrix, gates):
    n_tokens, d = x.shape
    w = sim_matrix[:, :EXPERTS]
    g = gates[:EXPERTS].reshape(1, EXPERTS)
    grid = (n_tokens // BLOCK_T,)
    return pl.pallas_call(
        _gate_body,
        grid=grid,
        in_specs=[
            pl.BlockSpec((BLOCK_T, d), lambda i: (i, 0)),
            pl.BlockSpec((d, EXPERTS), lambda i: (0, 0)),
            pl.BlockSpec((1, EXPERTS), lambda i: (0, 0)),
        ],
        out_specs=pl.BlockSpec((BLOCK_T, EXPERTS), lambda i: (i, 0)),
        out_shape=jax.ShapeDtypeStruct((n_tokens, EXPERTS), jnp.float32),
    )(x, w, g)
